# baseline (speedup 1.0000x reference)
import functools

import jax
import jax.numpy as jnp
from jax import lax
from jax.experimental import pallas as pl
from jax.experimental.pallas import tpu as pltpu
from jax.experimental.pallas import tpu_sc as plsc

N = 10000
E = 320000
D = 128
DE = 16

NC = 2
NS = 16
EW = E // (NC * NS)
C = 40
NCHUNK = EW // C
NBUF = 4
ZSTRIPE = 624
DRAIN = 80
NVEC = D // 16


def _tc_xw_body(x_ref, w_ref, xw_ref):
    xw_ref[...] = jnp.dot(x_ref[...], w_ref[...], preferred_element_type=jnp.float32)


def _tc_pe_body(e8_ref, w_ref, b_ref, p_ref):
    p = jnp.dot(e8_ref[...], w_ref[...],
                preferred_element_type=jnp.float32) + b_ref[...]
    p_ref[...] = p.reshape(p_ref.shape)


def _tc_out_body(mu0_ref, mu1_ref, mi0_ref, mi1_ref, w1_ref, w2_ref, b_ref, o_ref):
    mu = mu0_ref[...] + mu1_ref[...]
    mi = mi0_ref[...] + mi1_ref[...]
    acc = jnp.dot(mu, w1_ref[...], preferred_element_type=jnp.float32)
    acc = acc + jnp.dot(mi, w2_ref[...], preferred_element_type=jnp.float32)
    o_ref[...] = jnp.maximum(acc + b_ref[...], 0.0)


_MESH = plsc.VectorSubcoreMesh(core_axis_name="c", subcore_axis_name="s")

_SCRATCH = (
    [pltpu.VMEM((C,), jnp.int32)] * NBUF
    + [pltpu.VMEM((C,), jnp.int32)] * NBUF
    + [pltpu.VMEM((C, D), jnp.float32)] * NBUF
    + [pltpu.VMEM((C, D), jnp.float32)] * NBUF
    + [pltpu.SemaphoreType.DMA] * (5 * NBUF)
    + [pltpu.VMEM_SHARED((N, D), jnp.float32)]
)


@functools.partial(
    pl.kernel,
    out_type=[jax.ShapeDtypeStruct((N, D), jnp.float32),
              jax.ShapeDtypeStruct((N, D), jnp.float32)],
    mesh=_MESH,
    scratch_types=_SCRATCH,
)
def _sc_messages(xw, pe, src, dst, msg0, msg1, *rest):
    sbuf = rest[:NBUF]
    dbuf = rest[NBUF:2 * NBUF]
    rows = rest[2 * NBUF:3 * NBUF]
    pev = rest[3 * NBUF:4 * NBUF]
    semg = rest[4 * NBUF:5 * NBUF]
    semp = rest[5 * NBUF:6 * NBUF]
    sems = rest[6 * NBUF:7 * NBUF]
    semi = rest[7 * NBUF:8 * NBUF]
    semd = rest[8 * NBUF:9 * NBUF]
    acc = rest[9 * NBUF]

    cid = lax.axis_index("c")
    sid = lax.axis_index("s")
    tb = (cid * NS + sid) * EW

    zvec = jnp.zeros((16,), jnp.float32)

    def zbody(i, carry):
        for j in range(NVEC):
            rows[0][i, pl.ds(j * 16, 16)] = zvec
        return carry

    lax.fori_loop(0, DRAIN, zbody, 0)

    def zero_stripe(r0, total):
        nfull, rem = divmod(total, DRAIN)
        for k in range(nfull):
            pltpu.sync_copy(rows[0].at[pl.ds(0, DRAIN)],
                            acc.at[pl.ds(r0 + k * DRAIN, DRAIN)])
        if rem:
            pltpu.sync_copy(rows[0].at[pl.ds(0, rem)],
                            acc.at[pl.ds(r0 + nfull * DRAIN, rem)])

    @pl.when(sid < NS - 1)
    def _():
        zero_stripe(sid * ZSTRIPE, ZSTRIPE)

    @pl.when(sid == NS - 1)
    def _():
        zero_stripe((NS - 1) * ZSTRIPE, N - (NS - 1) * ZSTRIPE)

    plsc.subcore_barrier()

    pltpu.sync_copy(src.at[pl.ds(tb, C)], sbuf[0])
    pltpu.sync_copy(src.at[pl.ds(tb + C, C)], sbuf[1])
    pltpu.async_copy(src.at[pl.ds(tb + 2 * C, C)], sbuf[2], semi[2])
    pltpu.async_copy(src.at[pl.ds(tb + 3 * C, C)], sbuf[3], semi[3])
    pltpu.sync_copy(dst.at[pl.ds(tb, C)], dbuf[0])
    pltpu.sync_copy(dst.at[pl.ds(tb + C, C)], dbuf[1])
    pltpu.async_copy(dst.at[pl.ds(tb + 2 * C, C)], dbuf[2], semd[2])
    pltpu.async_copy(dst.at[pl.ds(tb + 3 * C, C)], dbuf[3], semd[3])

    def issue(j, b):
        pltpu.async_copy(xw.at[sbuf[b]], rows[b], semg[b])
        pltpu.async_copy(pe.at[pl.ds(tb + j * C, C)], pev[b], semp[b])

    issue(0, 0)
    issue(1, 1)

    def gbody(g, carry):
        for b in range(NBUF):
            j = g * NBUF + b
            bn = (b + 2) % NBUF

            @pl.when(j < NCHUNK)
            def _():
                pltpu.make_async_copy(xw.at[sbuf[b]], rows[b], semg[b]).wait()
                pltpu.make_async_copy(
                    pe.at[pl.ds(tb + j * C, C)], pev[b], semp[b]).wait()

            @pl.when(j + NBUF < NCHUNK)
            def _():
                pltpu.async_copy(
                    src.at[pl.ds(tb + (j + NBUF) * C, C)], sbuf[b], semi[b])

            @pl.when((j >= 2) & (j < NCHUNK + 2))
            def _():
                pltpu.make_async_copy(rows[bn], acc.at[dbuf[bn]], sems[bn]).wait()

            @pl.when((j >= 2) & (j + 2 < NCHUNK))
            def _():
                pltpu.async_copy(
                    dst.at[pl.ds(tb + (j + 2) * C, C)], dbuf[bn], semd[bn])

            @pl.when(j + 2 < NCHUNK)
            def _():
                pltpu.make_async_copy(
                    src.at[pl.ds(tb + (j + 2) * C, C)], sbuf[bn], semi[bn]).wait()
                issue(j + 2, bn)

            @pl.when(j < NCHUNK)
            def _():
                def ebody(i, c2):
                    for r in range(4):
                        ii = i * 4 + r
                        for j2 in range(NVEC):
                            s = pl.ds(j2 * 16, 16)
                            rows[b][ii, s] = jnp.maximum(
                                rows[b][ii, s] + pev[b][ii, s], 0.0)
                    return c2

                lax.fori_loop(0, C // 4, ebody, 0)

                @pl.when(j >= 2)
                def _():
                    pltpu.make_async_copy(
                        dst.at[pl.ds(tb + j * C, C)], dbuf[b], semd[b]).wait()

                pltpu.async_copy(rows[b], acc.at[dbuf[b]], sems[b], add=True)
        return carry

    lax.fori_loop(0, (NCHUNK + 2 + NBUF - 1) // NBUF, gbody, 0)
    plsc.subcore_barrier()

    def drain_stripe(msg_out, r0, total):
        nfull, rem = divmod(total, DRAIN)
        for k in range(nfull):
            o = r0 + k * DRAIN
            pltpu.sync_copy(acc.at[pl.ds(o, DRAIN)], rows[0].at[pl.ds(0, DRAIN)])
            pltpu.sync_copy(rows[0].at[pl.ds(0, DRAIN)], msg_out.at[pl.ds(o, DRAIN)])
        if rem:
            o = r0 + nfull * DRAIN
            pltpu.sync_copy(acc.at[pl.ds(o, rem)], rows[0].at[pl.ds(0, rem)])
            pltpu.sync_copy(rows[0].at[pl.ds(0, rem)], msg_out.at[pl.ds(o, rem)])

    def drain(msg_out):
        @pl.when(sid < NS - 1)
        def _():
            drain_stripe(msg_out, sid * ZSTRIPE, ZSTRIPE)

        @pl.when(sid == NS - 1)
        def _():
            drain_stripe(msg_out, (NS - 1) * ZSTRIPE, N - (NS - 1) * ZSTRIPE)

    @pl.when(cid == 0)
    def _():
        drain(msg0)

    @pl.when(cid == 1)
    def _():
        drain(msg1)


TB = 2560


def _pe_call(ea8, w_exp, b_exp):
    nb = E // TB
    return pl.pallas_call(
        _tc_pe_body,
        grid=(nb,),
        in_specs=[
            pl.BlockSpec((TB // 8, 8 * DE), lambda i: (i, 0)),
            pl.BlockSpec((8 * DE, 8 * D), lambda i: (0, 0)),
            pl.BlockSpec((1, 8 * D), lambda i: (0, 0)),
        ],
        out_specs=pl.BlockSpec((TB, D), lambda i: (i, 0)),
        out_shape=jax.ShapeDtypeStruct((E, D), jnp.float32),
    )(ea8, w_exp, b_exp)


def kernel(x, edge_index_user, edge_attr_user, edge_index_item, edge_attr_item,
           W_user, b_user, W_item, b_item, W_upd, b_upd):
    f32 = jnp.float32
    src_u, dst_u = edge_index_user[0], edge_index_user[1]
    src_i, dst_i = edge_index_item[0], edge_index_item[1]

    def _xw_call(w):
        return pl.pallas_call(
            _tc_xw_body,
            out_shape=jax.ShapeDtypeStruct((N, D), f32),
        )(x, w)

    xw_u = _xw_call(W_user[:D])

    eye8 = jnp.eye(8, dtype=f32)
    we_u = jnp.einsum('pq,kc->pkqc', eye8, W_user[D:]).reshape(8 * DE, 8 * D).astype(jnp.bfloat16)
    we_i = jnp.einsum('pq,kc->pkqc', eye8, W_item[D:]).reshape(8 * DE, 8 * D).astype(jnp.bfloat16)
    be_u = jnp.tile(b_user, (8,)).reshape(1, 8 * D)
    be_i = jnp.tile(b_item, (8,)).reshape(1, 8 * D)

    ea8_u = edge_attr_user.astype(jnp.bfloat16).reshape(E // 8, 8 * DE)
    ea8_i = edge_attr_item.astype(jnp.bfloat16).reshape(E // 8, 8 * DE)

    pe_u = _pe_call(ea8_u, we_u, be_u)
    mu0, mu1 = _sc_messages(xw_u, pe_u, src_u, dst_u)

    xw_i = _xw_call(W_item[:D])
    pe_i = _pe_call(ea8_i, we_i, be_i)
    mi0, mi1 = _sc_messages(xw_i, pe_i, src_i, dst_i)

    out = pl.pallas_call(
        _tc_out_body,
        out_shape=jax.ShapeDtypeStruct((N, D), f32),
    )(mu0, mu1, mi0, mi1, W_upd[:D], W_upd[D:], b_upd.reshape(1, D))
    return out


# speedup vs baseline: 1.0425x; 1.0425x over previous
import functools

import jax
import jax.numpy as jnp
from jax import lax
from jax.experimental import pallas as pl
from jax.experimental.pallas import tpu as pltpu
from jax.experimental.pallas import tpu_sc as plsc

N = 10000
E = 320000
D = 128
DE = 16

NC = 2
NS = 16
NSLAB = 2
EH = E // NSLAB
EW = EH // (NC * NS)
C = 40
NCHUNK = EW // C
NBUF = 4
ZSTRIPE = 624
DRAIN = 80
NVEC = D // 16


def _tc_xw_body(x_ref, w_ref, xw_ref):
    xw_ref[...] = jnp.dot(x_ref[...], w_ref[...], preferred_element_type=jnp.float32)


def _tc_pe_body(e8_ref, w_ref, b_ref, p_ref):
    p = jnp.dot(e8_ref[...], w_ref[...],
                preferred_element_type=jnp.float32) + b_ref[...]
    p_ref[...] = p.reshape(p_ref.shape)


def _tc_out_body(mu0_ref, mu1_ref, mu2_ref, mu3_ref,
                 mi0_ref, mi1_ref, mi2_ref, mi3_ref,
                 w1_ref, w2_ref, b_ref, o_ref):
    mu = (mu0_ref[...] + mu1_ref[...]) + (mu2_ref[...] + mu3_ref[...])
    mi = (mi0_ref[...] + mi1_ref[...]) + (mi2_ref[...] + mi3_ref[...])
    acc = jnp.dot(mu, w1_ref[...], preferred_element_type=jnp.float32)
    acc = acc + jnp.dot(mi, w2_ref[...], preferred_element_type=jnp.float32)
    o_ref[...] = jnp.maximum(acc + b_ref[...], 0.0)


_MESH = plsc.VectorSubcoreMesh(core_axis_name="c", subcore_axis_name="s")

_SCRATCH = (
    [pltpu.VMEM((C,), jnp.int32)] * NBUF
    + [pltpu.VMEM((C,), jnp.int32)] * NBUF
    + [pltpu.VMEM((C, D), jnp.float32)] * NBUF
    + [pltpu.VMEM((C, D), jnp.float32)] * NBUF
    + [pltpu.SemaphoreType.DMA] * (5 * NBUF)
    + [pltpu.VMEM_SHARED((N, D), jnp.float32)]
)


def _make_sc_messages(half):
    ix_off = half * EH

    @functools.partial(
        pl.kernel,
        out_type=[jax.ShapeDtypeStruct((N, D), jnp.float32),
                  jax.ShapeDtypeStruct((N, D), jnp.float32)],
        mesh=_MESH,
        scratch_types=_SCRATCH,
        name=f"sc_messages_h{half}",
    )
    def _sc_messages(xw, pe, src, dst, msg0, msg1, *rest):
        sbuf = rest[:NBUF]
        dbuf = rest[NBUF:2 * NBUF]
        rows = rest[2 * NBUF:3 * NBUF]
        pev = rest[3 * NBUF:4 * NBUF]
        semg = rest[4 * NBUF:5 * NBUF]
        semp = rest[5 * NBUF:6 * NBUF]
        sems = rest[6 * NBUF:7 * NBUF]
        semi = rest[7 * NBUF:8 * NBUF]
        semd = rest[8 * NBUF:9 * NBUF]
        acc = rest[9 * NBUF]

        cid = lax.axis_index("c")
        sid = lax.axis_index("s")
        tb = (cid * NS + sid) * EW
        tx = tb + ix_off

        zvec = jnp.zeros((16,), jnp.float32)

        def zbody(i, carry):
            for j in range(NVEC):
                rows[0][i, pl.ds(j * 16, 16)] = zvec
            return carry

        lax.fori_loop(0, DRAIN, zbody, 0)

        def zero_stripe(r0, total):
            nfull, rem = divmod(total, DRAIN)
            for k in range(nfull):
                pltpu.sync_copy(rows[0].at[pl.ds(0, DRAIN)],
                                acc.at[pl.ds(r0 + k * DRAIN, DRAIN)])
            if rem:
                pltpu.sync_copy(rows[0].at[pl.ds(0, rem)],
                                acc.at[pl.ds(r0 + nfull * DRAIN, rem)])

        @pl.when(sid < NS - 1)
        def _():
            zero_stripe(sid * ZSTRIPE, ZSTRIPE)

        @pl.when(sid == NS - 1)
        def _():
            zero_stripe((NS - 1) * ZSTRIPE, N - (NS - 1) * ZSTRIPE)

        plsc.subcore_barrier()

        pltpu.sync_copy(src.at[pl.ds(tx, C)], sbuf[0])
        pltpu.sync_copy(src.at[pl.ds(tx + C, C)], sbuf[1])
        pltpu.async_copy(src.at[pl.ds(tx + 2 * C, C)], sbuf[2], semi[2])
        pltpu.async_copy(src.at[pl.ds(tx + 3 * C, C)], sbuf[3], semi[3])
        pltpu.sync_copy(dst.at[pl.ds(tx, C)], dbuf[0])
        pltpu.sync_copy(dst.at[pl.ds(tx + C, C)], dbuf[1])
        pltpu.async_copy(dst.at[pl.ds(tx + 2 * C, C)], dbuf[2], semd[2])
        pltpu.async_copy(dst.at[pl.ds(tx + 3 * C, C)], dbuf[3], semd[3])

        def issue(j, b):
            pltpu.async_copy(xw.at[sbuf[b]], rows[b], semg[b])
            pltpu.async_copy(pe.at[pl.ds(tb + j * C, C)], pev[b], semp[b])

        issue(0, 0)
        issue(1, 1)

        def gbody(g, carry):
            for b in range(NBUF):
                j = g * NBUF + b
                bn = (b + 2) % NBUF

                @pl.when(j < NCHUNK)
                def _():
                    pltpu.make_async_copy(xw.at[sbuf[b]], rows[b], semg[b]).wait()
                    pltpu.make_async_copy(
                        pe.at[pl.ds(tb + j * C, C)], pev[b], semp[b]).wait()

                @pl.when(j + NBUF < NCHUNK)
                def _():
                    pltpu.async_copy(
                        src.at[pl.ds(tx + (j + NBUF) * C, C)], sbuf[b], semi[b])

                @pl.when((j >= 2) & (j < NCHUNK + 2))
                def _():
                    pltpu.make_async_copy(rows[bn], acc.at[dbuf[bn]], sems[bn]).wait()

                @pl.when((j >= 2) & (j + 2 < NCHUNK))
                def _():
                    pltpu.async_copy(
                        dst.at[pl.ds(tx + (j + 2) * C, C)], dbuf[bn], semd[bn])

                @pl.when(j + 2 < NCHUNK)
                def _():
                    pltpu.make_async_copy(
                        src.at[pl.ds(tx + (j + 2) * C, C)], sbuf[bn], semi[bn]).wait()
                    issue(j + 2, bn)

                @pl.when(j < NCHUNK)
                def _():
                    def ebody(i, c2):
                        for r in range(2):
                            ii = i * 2 + r
                            for j2 in range(NVEC):
                                s = pl.ds(j2 * 16, 16)
                                rows[b][ii, s] = jnp.maximum(
                                    rows[b][ii, s] + pev[b][ii, s], 0.0)
                        return c2

                    lax.fori_loop(0, C // 2, ebody, 0)

                    @pl.when(j >= 2)
                    def _():
                        pltpu.make_async_copy(
                            dst.at[pl.ds(tx + j * C, C)], dbuf[b], semd[b]).wait()

                    pltpu.async_copy(rows[b], acc.at[dbuf[b]], sems[b], add=True)
            return carry

        lax.fori_loop(0, (NCHUNK + 2 + NBUF - 1) // NBUF, gbody, 0)
        plsc.subcore_barrier()

        def drain_stripe(msg_out, r0, total):
            nfull, rem = divmod(total, DRAIN)
            for k in range(nfull):
                o = r0 + k * DRAIN
                pltpu.sync_copy(acc.at[pl.ds(o, DRAIN)], rows[0].at[pl.ds(0, DRAIN)])
                pltpu.sync_copy(rows[0].at[pl.ds(0, DRAIN)], msg_out.at[pl.ds(o, DRAIN)])
            if rem:
                o = r0 + nfull * DRAIN
                pltpu.sync_copy(acc.at[pl.ds(o, rem)], rows[0].at[pl.ds(0, rem)])
                pltpu.sync_copy(rows[0].at[pl.ds(0, rem)], msg_out.at[pl.ds(o, rem)])

        def drain(msg_out):
            @pl.when(sid < NS - 1)
            def _():
                drain_stripe(msg_out, sid * ZSTRIPE, ZSTRIPE)

            @pl.when(sid == NS - 1)
            def _():
                drain_stripe(msg_out, (NS - 1) * ZSTRIPE, N - (NS - 1) * ZSTRIPE)

        @pl.when(cid == 0)
        def _():
            drain(msg0)

        @pl.when(cid == 1)
        def _():
            drain(msg1)

    return _sc_messages


_SC_CALLS = [_make_sc_messages(h) for h in range(NSLAB)]


TB = 3200


def _pe_call(ea8, w_exp, b_exp, half):
    nb = EH // TB
    off = half * (EH // TB)
    return pl.pallas_call(
        _tc_pe_body,
        grid=(nb,),
        in_specs=[
            pl.BlockSpec((TB // 8, 8 * DE), lambda i: (i + off, 0)),
            pl.BlockSpec((8 * DE, 8 * D), lambda i: (0, 0)),
            pl.BlockSpec((1, 8 * D), lambda i: (0, 0)),
        ],
        out_specs=pl.BlockSpec((TB, D), lambda i: (i, 0)),
        out_shape=jax.ShapeDtypeStruct((EH, D), jnp.float32),
    )(ea8, w_exp, b_exp)


def kernel(x, edge_index_user, edge_attr_user, edge_index_item, edge_attr_item,
           W_user, b_user, W_item, b_item, W_upd, b_upd):
    f32 = jnp.float32
    src_u, dst_u = edge_index_user[0], edge_index_user[1]
    src_i, dst_i = edge_index_item[0], edge_index_item[1]

    def _xw_call(w):
        return pl.pallas_call(
            _tc_xw_body,
            out_shape=jax.ShapeDtypeStruct((N, D), f32),
        )(x, w)

    eye8 = jnp.eye(8, dtype=f32)
    we_u = jnp.einsum('pq,kc->pkqc', eye8, W_user[D:]).reshape(8 * DE, 8 * D).astype(jnp.bfloat16)
    we_i = jnp.einsum('pq,kc->pkqc', eye8, W_item[D:]).reshape(8 * DE, 8 * D).astype(jnp.bfloat16)
    be_u = jnp.tile(b_user, (8,)).reshape(1, 8 * D)
    be_i = jnp.tile(b_item, (8,)).reshape(1, 8 * D)

    ea8_u = edge_attr_user.astype(jnp.bfloat16).reshape(E // 8, 8 * DE)
    ea8_i = edge_attr_item.astype(jnp.bfloat16).reshape(E // 8, 8 * DE)

    xw_u = _xw_call(W_user[:D])
    msgs_u = []
    msgs_i = []

    pe_u0 = _pe_call(ea8_u, we_u, be_u, 0)
    msgs_u += _SC_CALLS[0](xw_u, pe_u0, src_u, dst_u)
    pe_u1 = _pe_call(ea8_u, we_u, be_u, 1)
    msgs_u += _SC_CALLS[1](xw_u, pe_u1, src_u, dst_u)

    xw_i = _xw_call(W_item[:D])
    pe_i0 = _pe_call(ea8_i, we_i, be_i, 0)
    msgs_i += _SC_CALLS[0](xw_i, pe_i0, src_i, dst_i)
    pe_i1 = _pe_call(ea8_i, we_i, be_i, 1)
    msgs_i += _SC_CALLS[1](xw_i, pe_i1, src_i, dst_i)

    out = pl.pallas_call(
        _tc_out_body,
        out_shape=jax.ShapeDtypeStruct((N, D), f32),
    )(*msgs_u, *msgs_i, W_upd[:D], W_upd[D:], b_upd.reshape(1, D))
    return out
